# baseline (speedup 1.0000x reference)
import functools

import jax
import jax.numpy as jnp
from jax import lax
from jax.experimental import pallas as pl
from jax.experimental.pallas import tpu as pltpu
from jax.experimental.pallas import tpu_sc as plsc

_N = 4194304
_D = 3
_TOTAL = _N * _D
_NC = 2
_NS = 16
_NW = _NC * _NS
_PER_W = _TOTAL // _NW
_CHUNK = 24576
_NCHUNK = _PER_W // _CHUNK
_GRP = 48

_mesh = plsc.VectorSubcoreMesh(core_axis_name="c", subcore_axis_name="s")


@functools.partial(
    pl.kernel,
    out_type=jax.ShapeDtypeStruct((_TOTAL,), jnp.float32),
    mesh=_mesh,
    scratch_types=[
        pltpu.VMEM((16,), jnp.float32),
        pltpu.VMEM((16,), jnp.float32),
        pltpu.VMEM((_CHUNK,), jnp.float32),
    ],
)
def _sc_shift(pos_hbm, pa_hbm, pb_hbm, out_hbm, pa_v, pb_v, buf):
    pltpu.sync_copy(pa_hbm, pa_v)
    pltpu.sync_copy(pb_hbm, pb_v)

    def _bf(x):
        u = lax.bitcast_convert_type(x, jnp.uint32)
        r = u + jnp.uint32(0x7FFF) + (
            lax.shift_right_logical(u, jnp.uint32(16)) & jnp.uint32(1))
        r = r & jnp.uint32(0xFFFF0000)
        return lax.bitcast_convert_type(r, jnp.float32)

    pa = pa_v[...]
    pb = pb_v[...]
    t = _bf(pa[0])
    p = _bf(pa[1])
    h = [
        jnp.maximum(_bf(pa[2 + 2 * j]) * t + _bf(pa[3 + 2 * j]) * p + pa[10 + j],
                    jnp.float32(0.0))
        for j in range(4)
    ]
    hb = [_bf(x) for x in h]
    s0 = (((_bf(pb[0]) * hb[0] + _bf(pb[1]) * hb[1]) + _bf(pb[2]) * hb[2])
          + _bf(pb[3]) * hb[3]) + pb[8]
    s2 = (((_bf(pb[4]) * hb[0] + _bf(pb[5]) * hb[1]) + _bf(pb[6]) * hb[2])
          + _bf(pb[7]) * hb[3]) + pb[9]
    zero_v = jnp.zeros((16,), jnp.float32)
    one_v = jnp.full((16,), 1.0, jnp.float32)
    s0_v = jnp.broadcast_to(s0, (16,))
    s2_v = jnp.broadcast_to(s2, (16,))
    pats = []
    for v in range(3):
        m = lax.rem(lax.iota(jnp.int32, 16) + jnp.int32(16 * v), jnp.int32(3))
        pat = jnp.where(m == 0, s0_v, jnp.where(m == 2, s2_v, zero_v))
        pat = lax.rem(pat, one_v)
        pat = jnp.where(pat < zero_v, pat + one_v, pat)
        pats.append(pat)

    wid = lax.axis_index("s") * _NC + lax.axis_index("c")
    base = wid * _PER_W

    def chunk_body(c, carry):
        off = base + c * _CHUNK
        pltpu.sync_copy(pos_hbm.at[pl.ds(off, _CHUNK)], buf)

        def grp(j, inner):
            for v in range(3):
                o = j * _GRP + v * 16
                x = buf[pl.ds(o, 16)]
                tt = x - pats[v]
                buf[pl.ds(o, 16)] = jnp.where(tt < zero_v, tt + one_v, tt)
            return inner

        lax.fori_loop(0, _CHUNK // _GRP, grp, 0)
        pltpu.sync_copy(buf, out_hbm.at[pl.ds(off, _CHUNK)])
        return carry

    lax.fori_loop(0, _NCHUNK, chunk_body, 0)


def kernel(pos, scale, temp, press, W1, b1, W2, b2):
    pa = jnp.concatenate([
        temp.reshape(1), press.reshape(1), W1.reshape(-1), b1.reshape(-1),
        jnp.zeros((2,), jnp.float32),
    ])
    pb = jnp.concatenate([
        W2.reshape(-1), b2.reshape(-1), jnp.zeros((6,), jnp.float32),
    ])
    out = _sc_shift(pos.reshape(-1), pa, pb)
    return (out.reshape(_N, _D), 0.0)


# speedup vs baseline: 25.9141x; 25.9141x over previous
import functools

import jax
import jax.numpy as jnp
from jax import lax
from jax.experimental import pallas as pl
from jax.experimental.pallas import tpu as pltpu
from jax.experimental.pallas import tpu_sc as plsc

_N = 4194304
_NC = 2
_NS = 16
_NW = _NC * _NS
_PER_W = 2 * _N // _NW
_CHUNK = 32768
_NCHUNK = _PER_W // _CHUNK

_mesh = plsc.VectorSubcoreMesh(core_axis_name="c", subcore_axis_name="s")


@functools.partial(
    pl.kernel,
    out_type=jax.ShapeDtypeStruct((2 * _N,), jnp.float32),
    mesh=_mesh,
    scratch_types=[
        pltpu.VMEM((16,), jnp.float32),
        pltpu.VMEM((16,), jnp.float32),
        pltpu.VMEM((_CHUNK,), jnp.float32),
    ],
)
def _sc_wrap(x_hbm, pa_hbm, pb_hbm, out_hbm, pa_v, pb_v, buf):
    pltpu.sync_copy(pa_hbm, pa_v)
    pltpu.sync_copy(pb_hbm, pb_v)

    def _bf(x):
        u = lax.bitcast_convert_type(x, jnp.uint32)
        r = u + jnp.uint32(0x7FFF) + (
            lax.shift_right_logical(u, jnp.uint32(16)) & jnp.uint32(1))
        r = r & jnp.uint32(0xFFFF0000)
        return lax.bitcast_convert_type(r, jnp.float32)

    pa = pa_v[...]
    pb = pb_v[...]
    t = _bf(pa[0])
    p = _bf(pa[1])
    h = [
        jnp.maximum(_bf(pa[2 + 2 * j]) * t + _bf(pa[3 + 2 * j]) * p + pa[10 + j],
                    jnp.float32(0.0))
        for j in range(4)
    ]
    hb = [_bf(x) for x in h]
    s0 = (((_bf(pb[0]) * hb[0] + _bf(pb[1]) * hb[1]) + _bf(pb[2]) * hb[2])
          + _bf(pb[3]) * hb[3]) + pb[8]
    s2 = (((_bf(pb[4]) * hb[0] + _bf(pb[5]) * hb[1]) + _bf(pb[6]) * hb[2])
          + _bf(pb[7]) * hb[3]) + pb[9]

    wid = lax.axis_index("s") * _NC + lax.axis_index("c")
    base = wid * _PER_W

    zero_v = jnp.zeros((16,), jnp.float32)
    one_v = jnp.full((16,), 1.0, jnp.float32)
    s_sel = jnp.where(wid < _NW // 2, s0, s2)
    sv = jnp.broadcast_to(s_sel, (16,))
    sv = lax.rem(sv, one_v)
    sv = jnp.where(sv < zero_v, sv + one_v, sv)

    def chunk_body(c, carry):
        off = base + c * _CHUNK
        pltpu.sync_copy(x_hbm.at[pl.ds(off, _CHUNK)], buf)

        def grp(j, inner):
            o = j * 16
            x = buf[pl.ds(o, 16)]
            tt = x - sv
            buf[pl.ds(o, 16)] = jnp.where(tt < zero_v, tt + one_v, tt)
            return inner

        lax.fori_loop(0, _CHUNK // 16, grp, 0)
        pltpu.sync_copy(buf, out_hbm.at[pl.ds(off, _CHUNK)])
        return carry

    lax.fori_loop(0, _NCHUNK, chunk_body, 0)


def kernel(pos, scale, temp, press, W1, b1, W2, b2):
    pa = jnp.concatenate([
        temp.reshape(1), press.reshape(1), W1.reshape(-1), b1.reshape(-1),
        jnp.zeros((2,), jnp.float32),
    ])
    pb = jnp.concatenate([
        W2.reshape(-1), b2.reshape(-1), jnp.zeros((6,), jnp.float32),
    ])
    pos_t = pos.T
    x02 = jnp.concatenate([pos_t[0], pos_t[2]])
    out02 = _sc_wrap(x02, pa, pb)
    new_pos = jnp.stack([out02[:_N], pos_t[1], out02[_N:]], axis=1)
    return (new_pos, 0.0)
